# baseline (speedup 1.0000x reference)
import jax
import jax.numpy as jnp
from jax import lax
from jax.experimental import pallas as pl
from jax.experimental.pallas import tpu as pltpu

NUM_PERM = 8
C = 8
N = 1024
D = 768

_ROWS = NUM_PERM * C * N
_BLOCK_ROWS = 1024


def _copy_body(in_ref, out_ref, argsort_ref, perm_ref):
    out_ref[...] = in_ref[...]
    argsort_ref[...] = lax.broadcasted_iota(jnp.int32, (NUM_PERM, N), 1)
    perm_ref[...] = lax.broadcasted_iota(jnp.int32, (1, NUM_PERM), 1)


_copy = pl.pallas_call(
    _copy_body,
    grid=(_ROWS // _BLOCK_ROWS,),
    in_specs=[pl.BlockSpec((_BLOCK_ROWS, D), lambda i: (i, 0))],
    out_specs=[
        pl.BlockSpec((_BLOCK_ROWS, D), lambda i: (i, 0)),
        pl.BlockSpec((NUM_PERM, N), lambda i: (0, 0)),
        pl.BlockSpec((1, NUM_PERM), lambda i: (0, 0)),
    ],
    out_shape=[
        jax.ShapeDtypeStruct((_ROWS, D), jnp.float32),
        jax.ShapeDtypeStruct((NUM_PERM, N), jnp.int32),
        jax.ShapeDtypeStruct((1, NUM_PERM), jnp.int32),
    ],
)


def kernel(patches):
    aug, argsort, perm2d = _copy(patches.reshape(_ROWS, D))
    return (aug.reshape(NUM_PERM, C, N, D), argsort, perm2d.reshape(NUM_PERM))


# speedup vs baseline: 1.0294x; 1.0294x over previous
import jax
import jax.numpy as jnp
from jax import lax
from jax.experimental import pallas as pl
from jax.experimental.pallas import tpu as pltpu

NUM_PERM = 8
C = 8
N = 1024
D = 768

_ROWS = NUM_PERM * C * N
_BLOCK_ROWS = 2048


def _copy_body(in_ref, out_ref, argsort_ref, perm_ref):
    out_ref[...] = in_ref[...]
    argsort_ref[...] = lax.broadcasted_iota(jnp.int32, (NUM_PERM, N), 1)
    perm_ref[...] = lax.broadcasted_iota(jnp.int32, (1, NUM_PERM), 1)


_copy = pl.pallas_call(
    _copy_body,
    grid=(_ROWS // _BLOCK_ROWS,),
    in_specs=[pl.BlockSpec((_BLOCK_ROWS, D), lambda i: (i, 0))],
    out_specs=[
        pl.BlockSpec((_BLOCK_ROWS, D), lambda i: (i, 0)),
        pl.BlockSpec((NUM_PERM, N), lambda i: (0, 0)),
        pl.BlockSpec((1, NUM_PERM), lambda i: (0, 0)),
    ],
    out_shape=[
        jax.ShapeDtypeStruct((_ROWS, D), jnp.float32),
        jax.ShapeDtypeStruct((NUM_PERM, N), jnp.int32),
        jax.ShapeDtypeStruct((1, NUM_PERM), jnp.int32),
    ],
)


def kernel(patches):
    aug, argsort, perm2d = _copy(patches.reshape(_ROWS, D))
    return (aug.reshape(NUM_PERM, C, N, D), argsort, perm2d.reshape(NUM_PERM))


# speedup vs baseline: 1.0373x; 1.0076x over previous
import jax
import jax.numpy as jnp
from jax import lax
from jax.experimental import pallas as pl
from jax.experimental.pallas import tpu as pltpu

NUM_PERM = 8
C = 8
N = 1024
D = 768

_ROWS = NUM_PERM * C * N
_BLOCK_ROWS = 4096


def _copy_body(in_ref, out_ref, argsort_ref, perm_ref):
    out_ref[...] = in_ref[...]
    argsort_ref[...] = lax.broadcasted_iota(jnp.int32, (NUM_PERM, N), 1)
    perm_ref[...] = lax.broadcasted_iota(jnp.int32, (1, NUM_PERM), 1)


_copy = pl.pallas_call(
    _copy_body,
    grid=(_ROWS // _BLOCK_ROWS,),
    in_specs=[pl.BlockSpec((_BLOCK_ROWS, D), lambda i: (i, 0))],
    out_specs=[
        pl.BlockSpec((_BLOCK_ROWS, D), lambda i: (i, 0)),
        pl.BlockSpec((NUM_PERM, N), lambda i: (0, 0)),
        pl.BlockSpec((1, NUM_PERM), lambda i: (0, 0)),
    ],
    out_shape=[
        jax.ShapeDtypeStruct((_ROWS, D), jnp.float32),
        jax.ShapeDtypeStruct((NUM_PERM, N), jnp.int32),
        jax.ShapeDtypeStruct((1, NUM_PERM), jnp.int32),
    ],
)


def kernel(patches):
    aug, argsort, perm2d = _copy(patches.reshape(_ROWS, D))
    return (aug.reshape(NUM_PERM, C, N, D), argsort, perm2d.reshape(NUM_PERM))
